# Initial kernel scaffold; baseline (speedup 1.0000x reference)
import jax
import jax.numpy as jnp
from jax.experimental import pallas as pl


def kernel(x, edge_index, W_in, b_in, W1, b1, W2, b2, We, be):
    raise NotImplementedError("write your pallas kernel here")


# speedup vs baseline: 4.6903x; 4.6903x over previous
import jax
import jax.numpy as jnp
from jax import lax
from jax.experimental import pallas as pl
from jax.experimental.pallas import tpu as pltpu
from jax.experimental.pallas import tpu_sc as plsc

N = 10000
E = 320000
D_IN = 128
D_H = 8
D_OUT = 4
NC = 2
NS = 16
NW = NC * NS
EPW = E // NW
K = 80
NCHUNK = EPW // K
NPAD = 10240
RPT = NPAD // NS


def _node_pre_body(x_ref, win_ref, bin_ref, w1_ref, b1_ref, w2_ref, b2_ref,
                   we_ref, be_ref, p_ref, q_ref, rsx_ref):
    x = x_ref[...]
    w2 = w2_ref[...]
    wc = jnp.dot(w1_ref[...], w2, preferred_element_type=jnp.float32)
    bc = jnp.dot(b1_ref[...], w2, preferred_element_type=jnp.float32) + b2_ref[...]
    h = jnp.dot(x, win_ref[...], preferred_element_type=jnp.float32) + bin_ref[...]
    h = jnp.where(h > 0, h, jnp.exp(h) - 1.0)
    hx = jnp.concatenate([h, x], axis=1)
    p = jnp.dot(hx, wc[:136] - wc[136:], preferred_element_type=jnp.float32) + bc
    q = jnp.dot(hx, wc[136:], preferred_element_type=jnp.float32)
    we = we_ref[...]
    rx = jnp.dot(x, we[8:136], preferred_element_type=jnp.float32)
    sx = jnp.dot(x, we[144:], preferred_element_type=jnp.float32) + be_ref[...]
    pad = jnp.zeros((NPAD - N, D_H), jnp.float32)
    p_ref[...] = jnp.concatenate([p, pad], axis=0)
    q_ref[...] = jnp.concatenate([q, pad], axis=0)
    rsx_ref[...] = jnp.concatenate([jnp.concatenate([rx, sx], axis=1), pad],
                                   axis=0)


_node_pre = pl.pallas_call(
    _node_pre_body,
    out_shape=[
        jax.ShapeDtypeStruct((NPAD, D_H), jnp.float32),
        jax.ShapeDtypeStruct((NPAD, D_H), jnp.float32),
        jax.ShapeDtypeStruct((NPAD, 2 * D_OUT), jnp.float32),
    ],
)


def _edge_msg_body(p_hbm, q_hbm, src_hbm, dst_hbm, zero_hbm, out_hbm,
                   dv, sv, pv, qv, acc, sem):
    c = lax.axis_index("c")
    s = lax.axis_index("s")
    wid = c * NS + s
    sl = pl.ds(s * RPT, RPT)
    pltpu.sync_copy(zero_hbm.at[sl], acc.at[sl])
    plsc.subcore_barrier()

    iota = lax.iota(jnp.int32, 16)
    rowp = lax.shift_right_arithmetic(iota, jnp.int32(3))
    colp = lax.bitwise_and(iota, jnp.int32(7))
    base0 = wid * EPW

    def chunk(i, carry):
        base = base0 + i * K
        pltpu.sync_copy(dst_hbm.at[pl.ds(base, K)], dv)
        pltpu.sync_copy(src_hbm.at[pl.ds(base, K)], sv)
        pltpu.async_copy(p_hbm.at[dv], pv, sem).wait()
        pltpu.async_copy(q_hbm.at[sv], qv, sem).wait()
        for j in range(K * D_H // 16):
            ridx = rowp + 2 * j
            pvec = plsc.load_gather(pv, [ridx, colp])
            qvec = plsc.load_gather(qv, [ridx, colp])
            v = pvec + qvec
            t = jnp.where(v > 0, v, jnp.exp(v) - 1.0)
            plsc.store_scatter(pv, [ridx, colp], t)
        pltpu.sync_copy(pv, acc.at[dv], add=True)
        return carry

    lax.fori_loop(0, NCHUNK, chunk, 0)
    plsc.subcore_barrier()
    pltpu.sync_copy(acc.at[sl], out_hbm.at[pl.ds(c * NPAD + s * RPT, RPT)])


def _node_out_body(hp_ref, rsx_ref, we_ref, rs_ref):
    hsum = hp_ref[:N] + hp_ref[NPAD:NPAD + N]
    we = we_ref[...]
    r = jnp.dot(hsum, we[0:8], preferred_element_type=jnp.float32)
    s = jnp.dot(hsum, we[136:144], preferred_element_type=jnp.float32)
    rs = jnp.concatenate([r, s], axis=1) + rsx_ref[:N]
    pad = jnp.zeros((NPAD - N, 2 * D_OUT), jnp.float32)
    rs_ref[...] = jnp.concatenate([rs, pad], axis=0)


_node_out = pl.pallas_call(
    _node_out_body,
    out_shape=[jax.ShapeDtypeStruct((NPAD, 2 * D_OUT), jnp.float32)],
)


def _edge_out_body(rs_hbm, src_hbm, dst_hbm, eo_hbm, sv, dv, rb, db, ov, sem):
    c = lax.axis_index("c")
    s = lax.axis_index("s")
    wid = c * NS + s
    iota = lax.iota(jnp.int32, 16)
    e4 = lax.shift_right_arithmetic(iota, jnp.int32(2))
    c4 = lax.bitwise_and(iota, jnp.int32(3))
    base0 = wid * EPW

    def chunk(i, carry):
        base = base0 + i * K
        pltpu.sync_copy(src_hbm.at[pl.ds(base, K)], sv)
        pltpu.sync_copy(dst_hbm.at[pl.ds(base, K)], dv)
        pltpu.async_copy(rs_hbm.at[sv], rb, sem).wait()
        pltpu.async_copy(rs_hbm.at[dv], db, sem).wait()
        for g in range(K // 4):
            ridx = e4 + 4 * g
            rvec = plsc.load_gather(rb, [ridx, c4])
            svec = plsc.load_gather(db, [ridx, c4 + 4])
            ov[pl.ds(g * 16, 16)] = rvec + svec
        pltpu.sync_copy(ov, eo_hbm.at[pl.ds(base * D_OUT, K * D_OUT)])
        return carry

    lax.fori_loop(0, NCHUNK, chunk, 0)


def _make_sc_kernels():
    mesh = plsc.VectorSubcoreMesh(core_axis_name="c", subcore_axis_name="s")
    cp = pltpu.CompilerParams(needs_layout_passes=False,
                              use_tc_tiling_on_sc=False)
    edge_msg = pl.kernel(
        _edge_msg_body,
        out_type=jax.ShapeDtypeStruct((NC * NPAD, D_H), jnp.float32),
        mesh=mesh,
        compiler_params=cp,
        scratch_types=[
            pltpu.VMEM((K,), jnp.int32),
            pltpu.VMEM((K,), jnp.int32),
            pltpu.VMEM((K, D_H), jnp.float32),
            pltpu.VMEM((K, D_H), jnp.float32),
            pltpu.VMEM_SHARED((NPAD, D_H), jnp.float32),
            pltpu.SemaphoreType.DMA,
        ],
    )
    edge_out = pl.kernel(
        _edge_out_body,
        out_type=jax.ShapeDtypeStruct((E * D_OUT,), jnp.float32),
        mesh=mesh,
        compiler_params=cp,
        scratch_types=[
            pltpu.VMEM((K,), jnp.int32),
            pltpu.VMEM((K,), jnp.int32),
            pltpu.VMEM((K, 2 * D_OUT), jnp.float32),
            pltpu.VMEM((K, 2 * D_OUT), jnp.float32),
            pltpu.VMEM((K * D_OUT,), jnp.float32),
            pltpu.SemaphoreType.DMA,
        ],
    )
    return edge_msg, edge_out


def kernel(x, edge_index, W_in, b_in, W1, b1, W2, b2, We, be):
    edge_msg, edge_out = _make_sc_kernels()
    src = edge_index[0]
    dst = edge_index[1]
    p, q, rsx = _node_pre(x, W_in, b_in.reshape(1, -1), W1, b1.reshape(1, -1),
                          W2, b2.reshape(1, -1), We, be.reshape(1, -1))
    zeros = jnp.zeros((NPAD, D_H), jnp.float32)
    hp = edge_msg(p, q, src, dst, zeros)
    (rs,) = _node_out(hp, rsx, We)
    eo = edge_out(rs, src, dst)
    return eo.reshape(E, D_OUT)


# speedup vs baseline: 8.0219x; 1.7103x over previous
import jax
import jax.numpy as jnp
from jax import lax
from jax.experimental import pallas as pl
from jax.experimental.pallas import tpu as pltpu
from jax.experimental.pallas import tpu_sc as plsc

N = 10000
E = 320000
D_IN = 128
D_H = 8
D_OUT = 4
NC = 2
NS = 16
NW = NC * NS
EPW = E // NW
K = 80
NCHUNK = EPW // K
NPAD = 10240
RPT = NPAD // NS
KB = 128
NFULL = EPW // KB
NPAIR = NFULL // 2
KT = EPW - NFULL * KB
TBASE = NFULL * KB


def _node_pre_body(x_ref, win_ref, bin_ref, w1_ref, b1_ref, w2_ref, b2_ref,
                   we_ref, be_ref, p_ref, q_ref, rsx_ref):
    x = x_ref[...]
    w2 = w2_ref[...]
    wc = jnp.dot(w1_ref[...], w2, preferred_element_type=jnp.float32)
    bc = jnp.dot(b1_ref[...], w2, preferred_element_type=jnp.float32) + b2_ref[...]
    h = jnp.dot(x, win_ref[...], preferred_element_type=jnp.float32) + bin_ref[...]
    h = jnp.where(h > 0, h, jnp.exp(h) - 1.0)
    hx = jnp.concatenate([h, x], axis=1)
    p = jnp.dot(hx, wc[:136] - wc[136:], preferred_element_type=jnp.float32) + bc
    q = jnp.dot(hx, wc[136:], preferred_element_type=jnp.float32)
    we = we_ref[...]
    rx = jnp.dot(x, we[8:136], preferred_element_type=jnp.float32)
    sx = jnp.dot(x, we[144:], preferred_element_type=jnp.float32) + be_ref[...]
    pad = jnp.zeros((NPAD - N, D_H), jnp.float32)
    p_ref[...] = jnp.concatenate([p, pad], axis=0)
    q_ref[...] = jnp.concatenate([q, pad], axis=0)
    rsx_ref[...] = jnp.concatenate([jnp.concatenate([rx, sx], axis=1), pad],
                                   axis=0)


_node_pre = pl.pallas_call(
    _node_pre_body,
    out_shape=[
        jax.ShapeDtypeStruct((NPAD, D_H), jnp.float32),
        jax.ShapeDtypeStruct((NPAD, D_H), jnp.float32),
        jax.ShapeDtypeStruct((NPAD, 2 * D_OUT), jnp.float32),
    ],
)


def _msg_compute(pv, qv, rowp, colp, ngroups):
    for j in range(ngroups):
        ridx = rowp + 2 * j
        pvec = plsc.load_gather(pv, [ridx, colp])
        qvec = plsc.load_gather(qv, [ridx, colp])
        v = pvec + qvec
        t = jnp.where(v > 0, v, jnp.exp(v) - 1.0)
        plsc.store_scatter(pv, [ridx, colp], t)


def _edge_msg_body(p_hbm, q_hbm, ef_hbm, zero_hbm, out_hbm,
                   ivd, ivs, pva, qva, pvb, qvb, pt, qt, acc,
                   semg_a, semg_b, sems_a, sems_b):
    c = lax.axis_index("c")
    s = lax.axis_index("s")
    wid = c * NS + s
    sl = pl.ds(s * RPT, RPT)
    pltpu.sync_copy(zero_hbm.at[sl], acc.at[sl])
    plsc.subcore_barrier()

    iota = lax.iota(jnp.int32, 16)
    rowp = lax.shift_right_arithmetic(iota, jnp.int32(3))
    colp = lax.bitwise_and(iota, jnp.int32(7))
    base0 = wid * EPW

    def pair(t, carry):
        base = base0 + t * (2 * KB)
        pltpu.sync_copy(ef_hbm.at[pl.ds(E + base, 2 * KB)], ivd)
        pltpu.sync_copy(ef_hbm.at[pl.ds(base, 2 * KB)], ivs)
        da = ivd.at[pl.ds(0, KB)]
        db = ivd.at[pl.ds(KB, KB)]
        sa = ivs.at[pl.ds(0, KB)]
        sb = ivs.at[pl.ds(KB, KB)]
        hpa = pltpu.async_copy(p_hbm.at[da], pva, semg_a)
        hqa = pltpu.async_copy(q_hbm.at[sa], qva, semg_a)
        hpb = pltpu.async_copy(p_hbm.at[db], pvb, semg_b)
        hqb = pltpu.async_copy(q_hbm.at[sb], qvb, semg_b)
        hpa.wait()
        hqa.wait()
        _msg_compute(pva, qva, rowp, colp, KB * D_H // 16)
        ssa = pltpu.async_copy(pva, acc.at[da], sems_a, add=True)
        hpb.wait()
        hqb.wait()
        _msg_compute(pvb, qvb, rowp, colp, KB * D_H // 16)
        ssb = pltpu.async_copy(pvb, acc.at[db], sems_b, add=True)
        ssa.wait()
        ssb.wait()
        return carry

    lax.fori_loop(0, NPAIR, pair, 0)

    tb = base0 + TBASE
    dt = ivd.at[pl.ds(0, KT)]
    st = ivs.at[pl.ds(0, KT)]
    pltpu.sync_copy(ef_hbm.at[pl.ds(E + tb, KT)], dt)
    pltpu.sync_copy(ef_hbm.at[pl.ds(tb, KT)], st)
    hp = pltpu.async_copy(p_hbm.at[dt], pt, semg_a)
    hq = pltpu.async_copy(q_hbm.at[st], qt, semg_a)
    hp.wait()
    hq.wait()
    _msg_compute(pt, qt, rowp, colp, KT * D_H // 16)
    pltpu.sync_copy(pt, acc.at[dt], add=True)

    plsc.subcore_barrier()
    pltpu.sync_copy(acc.at[sl], out_hbm.at[pl.ds(c * NPAD + s * RPT, RPT)])


def _node_out_body(hp_ref, rsx_ref, we_ref, rs_ref):
    hsum = hp_ref[:N] + hp_ref[NPAD:NPAD + N]
    we = we_ref[...]
    r = jnp.dot(hsum, we[0:8], preferred_element_type=jnp.float32)
    s = jnp.dot(hsum, we[136:144], preferred_element_type=jnp.float32)
    rs = jnp.concatenate([r, s], axis=1) + rsx_ref[:N]
    pad = jnp.zeros((NPAD - N, 2 * D_OUT), jnp.float32)
    rs_ref[...] = jnp.concatenate([rs, pad], axis=0)


_node_out = pl.pallas_call(
    _node_out_body,
    out_shape=[jax.ShapeDtypeStruct((NPAD, 2 * D_OUT), jnp.float32)],
)


def _out_compute(rb, db, ov, e4, c4, ngroups):
    for g in range(ngroups):
        ridx = e4 + 4 * g
        rvec = plsc.load_gather(rb, [ridx, c4])
        svec = plsc.load_gather(db, [ridx, c4 + 4])
        ov[pl.ds(g * 16, 16)] = rvec + svec


def _edge_out_body(rs_hbm, ef_hbm, eo_hbm, ivs, ivd, ra, da, rb, db,
                   ova, ovb, rt, dt, semg_a, semg_b, semo_a, semo_b):
    c = lax.axis_index("c")
    s = lax.axis_index("s")
    wid = c * NS + s
    iota = lax.iota(jnp.int32, 16)
    e4 = lax.shift_right_arithmetic(iota, jnp.int32(2))
    c4 = lax.bitwise_and(iota, jnp.int32(3))
    base0 = wid * EPW

    def pair(t, carry):
        base = base0 + t * (2 * KB)
        pltpu.sync_copy(ef_hbm.at[pl.ds(base, 2 * KB)], ivs)
        pltpu.sync_copy(ef_hbm.at[pl.ds(E + base, 2 * KB)], ivd)
        sa = ivs.at[pl.ds(0, KB)]
        sb = ivs.at[pl.ds(KB, KB)]
        dac = ivd.at[pl.ds(0, KB)]
        dbc = ivd.at[pl.ds(KB, KB)]
        hra = pltpu.async_copy(rs_hbm.at[sa], ra, semg_a)
        hda = pltpu.async_copy(rs_hbm.at[dac], da, semg_a)
        hrb = pltpu.async_copy(rs_hbm.at[sb], rb, semg_b)
        hdb = pltpu.async_copy(rs_hbm.at[dbc], db, semg_b)
        hra.wait()
        hda.wait()
        _out_compute(ra, da, ova, e4, c4, KB // 4)
        hoa = pltpu.async_copy(ova, eo_hbm.at[pl.ds(base * D_OUT, KB * D_OUT)],
                               semo_a)
        hrb.wait()
        hdb.wait()
        _out_compute(rb, db, ovb, e4, c4, KB // 4)
        hob = pltpu.async_copy(
            ovb, eo_hbm.at[pl.ds((base + KB) * D_OUT, KB * D_OUT)], semo_b)
        hoa.wait()
        hob.wait()
        return carry

    lax.fori_loop(0, NPAIR, pair, 0)

    tb = base0 + TBASE
    st = ivs.at[pl.ds(0, KT)]
    dtc = ivd.at[pl.ds(0, KT)]
    pltpu.sync_copy(ef_hbm.at[pl.ds(tb, KT)], st)
    pltpu.sync_copy(ef_hbm.at[pl.ds(E + tb, KT)], dtc)
    hr = pltpu.async_copy(rs_hbm.at[st], rt, semg_a)
    hd = pltpu.async_copy(rs_hbm.at[dtc], dt, semg_a)
    hr.wait()
    hd.wait()
    _out_compute(rt, dt, ova, e4, c4, KT // 4)
    pltpu.sync_copy(ova.at[pl.ds(0, KT * D_OUT)],
                    eo_hbm.at[pl.ds(tb * D_OUT, KT * D_OUT)])


def _make_sc_kernels():
    mesh = plsc.VectorSubcoreMesh(core_axis_name="c", subcore_axis_name="s")
    cp = pltpu.CompilerParams(needs_layout_passes=False,
                              use_tc_tiling_on_sc=False)
    edge_msg = pl.kernel(
        _edge_msg_body,
        out_type=jax.ShapeDtypeStruct((NC * NPAD, D_H), jnp.float32),
        mesh=mesh,
        compiler_params=cp,
        scratch_types=[
            pltpu.VMEM((2 * KB,), jnp.int32),
            pltpu.VMEM((2 * KB,), jnp.int32),
            pltpu.VMEM((KB, D_H), jnp.float32),
            pltpu.VMEM((KB, D_H), jnp.float32),
            pltpu.VMEM((KB, D_H), jnp.float32),
            pltpu.VMEM((KB, D_H), jnp.float32),
            pltpu.VMEM((KT, D_H), jnp.float32),
            pltpu.VMEM((KT, D_H), jnp.float32),
            pltpu.VMEM_SHARED((NPAD, D_H), jnp.float32),
            pltpu.SemaphoreType.DMA,
            pltpu.SemaphoreType.DMA,
            pltpu.SemaphoreType.DMA,
            pltpu.SemaphoreType.DMA,
        ],
    )
    edge_out = pl.kernel(
        _edge_out_body,
        out_type=jax.ShapeDtypeStruct((E * D_OUT,), jnp.float32),
        mesh=mesh,
        compiler_params=cp,
        scratch_types=[
            pltpu.VMEM((2 * KB,), jnp.int32),
            pltpu.VMEM((2 * KB,), jnp.int32),
            pltpu.VMEM((KB, 2 * D_OUT), jnp.float32),
            pltpu.VMEM((KB, 2 * D_OUT), jnp.float32),
            pltpu.VMEM((KB, 2 * D_OUT), jnp.float32),
            pltpu.VMEM((KB, 2 * D_OUT), jnp.float32),
            pltpu.VMEM((KB * D_OUT,), jnp.float32),
            pltpu.VMEM((KB * D_OUT,), jnp.float32),
            pltpu.VMEM((KT, 2 * D_OUT), jnp.float32),
            pltpu.VMEM((KT, 2 * D_OUT), jnp.float32),
            pltpu.SemaphoreType.DMA,
            pltpu.SemaphoreType.DMA,
            pltpu.SemaphoreType.DMA,
            pltpu.SemaphoreType.DMA,
        ],
    )
    return edge_msg, edge_out


def kernel(x, edge_index, W_in, b_in, W1, b1, W2, b2, We, be):
    edge_msg, edge_out = _make_sc_kernels()
    ef = edge_index.reshape(2 * E)
    p, q, rsx = _node_pre(x, W_in, b_in.reshape(1, -1), W1, b1.reshape(1, -1),
                          W2, b2.reshape(1, -1), We, be.reshape(1, -1))
    zeros = jnp.zeros((NPAD, D_H), jnp.float32)
    hp = edge_msg(p, q, ef, zeros)
    (rs,) = _node_out(hp, rsx, We)
    eo = edge_out(rs, ef)
    return eo.reshape(E, D_OUT)


# speedup vs baseline: 8.0546x; 1.0041x over previous
import jax
import jax.numpy as jnp
from jax import lax
from jax.experimental import pallas as pl
from jax.experimental.pallas import tpu as pltpu
from jax.experimental.pallas import tpu_sc as plsc

N = 10000
E = 320000
D_IN = 128
D_H = 8
D_OUT = 4
NC = 2
NS = 16
NW = NC * NS
EPW = E // NW
K = 80
NCHUNK = EPW // K
NPAD = 10240
RPT = NPAD // NS
KB = 128
NFULL = EPW // KB
NPAIR = NFULL // 2
KT = EPW - NFULL * KB
TBASE = NFULL * KB


def _node_pre_body(x_ref, win_ref, bin_ref, w1_ref, b1_ref, w2_ref, b2_ref,
                   we_ref, be_ref, p_ref, q_ref, rsx_ref):
    x = x_ref[...]
    w2 = w2_ref[...]
    wc = jnp.dot(w1_ref[...], w2, preferred_element_type=jnp.float32)
    bc = jnp.dot(b1_ref[...], w2, preferred_element_type=jnp.float32) + b2_ref[...]
    h = jnp.dot(x, win_ref[...], preferred_element_type=jnp.float32) + bin_ref[...]
    h = jnp.where(h > 0, h, jnp.exp(h) - 1.0)
    hx = jnp.concatenate([h, x], axis=1)
    p = jnp.dot(hx, wc[:136] - wc[136:], preferred_element_type=jnp.float32) + bc
    q = jnp.dot(hx, wc[136:], preferred_element_type=jnp.float32)
    we = we_ref[...]
    rx = jnp.dot(x, we[8:136], preferred_element_type=jnp.float32)
    sx = jnp.dot(x, we[144:], preferred_element_type=jnp.float32) + be_ref[...]
    pad = jnp.zeros((NPAD - N, D_H), jnp.float32)
    p_ref[...] = jnp.concatenate([p, pad], axis=0)
    q_ref[...] = jnp.concatenate([q, pad], axis=0)
    rsx_ref[...] = jnp.concatenate([jnp.concatenate([rx, sx], axis=1), pad],
                                   axis=0)


_node_pre = pl.pallas_call(
    _node_pre_body,
    out_shape=[
        jax.ShapeDtypeStruct((NPAD, D_H), jnp.float32),
        jax.ShapeDtypeStruct((NPAD, D_H), jnp.float32),
        jax.ShapeDtypeStruct((NPAD, 2 * D_OUT), jnp.float32),
    ],
)


def _msg_compute(pv, qv, rowp, colp, ngroups):
    for j in range(ngroups):
        ridx = rowp + 2 * j
        pvec = plsc.load_gather(pv, [ridx, colp])
        qvec = plsc.load_gather(qv, [ridx, colp])
        v = pvec + qvec
        t = jnp.where(v > 0, v, jnp.exp(v) - 1.0)
        plsc.store_scatter(pv, [ridx, colp], t)


def _edge_msg_body(p_hbm, q_hbm, ef_hbm, zero_hbm, out_hbm,
                   ivd, ivs, pva, qva, pvb, qvb, pt, qt, acc,
                   semg_a, semg_b, sems_a, sems_b):
    c = lax.axis_index("c")
    s = lax.axis_index("s")
    wid = c * NS + s
    sl = pl.ds(s * RPT, RPT)
    pltpu.sync_copy(zero_hbm.at[sl], acc.at[sl])
    plsc.subcore_barrier()

    iota = lax.iota(jnp.int32, 16)
    rowp = lax.shift_right_arithmetic(iota, jnp.int32(3))
    colp = lax.bitwise_and(iota, jnp.int32(7))
    base0 = wid * EPW

    def pair(t, carry):
        base = base0 + t * (2 * KB)
        pltpu.sync_copy(ef_hbm.at[pl.ds(E + base, 2 * KB)], ivd)
        pltpu.sync_copy(ef_hbm.at[pl.ds(base, 2 * KB)], ivs)
        da = ivd.at[pl.ds(0, KB)]
        db = ivd.at[pl.ds(KB, KB)]
        sa = ivs.at[pl.ds(0, KB)]
        sb = ivs.at[pl.ds(KB, KB)]
        hpa = pltpu.async_copy(p_hbm.at[da], pva, semg_a)
        hqa = pltpu.async_copy(q_hbm.at[sa], qva, semg_a)
        hpb = pltpu.async_copy(p_hbm.at[db], pvb, semg_b)
        hqb = pltpu.async_copy(q_hbm.at[sb], qvb, semg_b)
        hpa.wait()
        hqa.wait()
        _msg_compute(pva, qva, rowp, colp, KB * D_H // 16)
        ssa = pltpu.async_copy(pva, acc.at[da], sems_a, add=True)
        hpb.wait()
        hqb.wait()
        _msg_compute(pvb, qvb, rowp, colp, KB * D_H // 16)
        ssb = pltpu.async_copy(pvb, acc.at[db], sems_b, add=True)
        ssa.wait()
        ssb.wait()
        return carry

    lax.fori_loop(0, NPAIR, pair, 0)

    tb = base0 + TBASE
    dt = ivd.at[pl.ds(0, KT)]
    st = ivs.at[pl.ds(0, KT)]
    pltpu.sync_copy(ef_hbm.at[pl.ds(E + tb, KT)], dt)
    pltpu.sync_copy(ef_hbm.at[pl.ds(tb, KT)], st)
    hp = pltpu.async_copy(p_hbm.at[dt], pt, semg_a)
    hq = pltpu.async_copy(q_hbm.at[st], qt, semg_a)
    hp.wait()
    hq.wait()
    _msg_compute(pt, qt, rowp, colp, KT * D_H // 16)
    pltpu.sync_copy(pt, acc.at[dt], add=True)

    plsc.subcore_barrier()
    pltpu.sync_copy(acc.at[sl], out_hbm.at[pl.ds(c * NPAD + s * RPT, RPT)])


def _out_compute(rb, db, ov, e4, c4, ngroups):
    for g in range(ngroups):
        ridx = e4 + 4 * g
        rvec = plsc.load_gather(rb, [ridx, c4])
        svec = plsc.load_gather(db, [ridx, c4 + 4])
        ov[pl.ds(g * 16, 16)] = rvec + svec


def _edge_out_body(hp_hbm, rsx_hbm, wcat_hbm, ef_hbm, eo_hbm, rs_hbm,
                   wv, ivs, ivd, ra, da, rb, db,
                   ova, ovb, rt, dt, semg_a, semg_b, semo_a, semo_b):
    c = lax.axis_index("c")
    s = lax.axis_index("s")
    wid = c * NS + s
    iota = lax.iota(jnp.int32, 16)
    rowp = lax.shift_right_arithmetic(iota, jnp.int32(3))
    colp = lax.bitwise_and(iota, jnp.int32(7))
    e4 = lax.shift_right_arithmetic(iota, jnp.int32(2))
    c4 = lax.bitwise_and(iota, jnp.int32(3))
    base0 = wid * EPW

    pltpu.sync_copy(wcat_hbm, wv)
    wks = [plsc.load_gather(wv, [jnp.full((16,), k, jnp.int32), colp])
           for k in range(8)]
    r0 = s * RPT

    def rs_block(i, carry):
        rr = r0 + i * KB
        h0 = pltpu.async_copy(hp_hbm.at[pl.ds(rr, KB)], ra, semg_a)
        h1 = pltpu.async_copy(hp_hbm.at[pl.ds(NPAD + rr, KB)], da, semg_b)
        hx = pltpu.async_copy(rsx_hbm.at[pl.ds(rr, KB)], rb, semo_a)
        h0.wait()
        h1.wait()
        for j in range(KB * D_H // 16):
            ridx = rowp + 2 * j
            v = plsc.load_gather(ra, [ridx, colp]) + \
                plsc.load_gather(da, [ridx, colp])
            plsc.store_scatter(ra, [ridx, colp], v)
        hx.wait()
        for g in range(KB * D_H // 16):
            nid = rowp + 2 * g
            acc = plsc.load_gather(rb, [nid, colp])
            for k in range(8):
                hk = plsc.load_gather(ra, [nid, jnp.full((16,), k, jnp.int32)])
                acc = acc + hk * wks[k]
            plsc.store_scatter(rb, [nid, colp], acc)
        pltpu.sync_copy(rb, rs_hbm.at[pl.ds(rr, KB)])
        return carry

    lax.fori_loop(0, RPT // KB, rs_block, 0)
    plsc.subcore_barrier()


    def pair(t, carry):
        base = base0 + t * (2 * KB)
        pltpu.sync_copy(ef_hbm.at[pl.ds(base, 2 * KB)], ivs)
        pltpu.sync_copy(ef_hbm.at[pl.ds(E + base, 2 * KB)], ivd)
        sa = ivs.at[pl.ds(0, KB)]
        sb = ivs.at[pl.ds(KB, KB)]
        dac = ivd.at[pl.ds(0, KB)]
        dbc = ivd.at[pl.ds(KB, KB)]
        hra = pltpu.async_copy(rs_hbm.at[sa], ra, semg_a)
        hda = pltpu.async_copy(rs_hbm.at[dac], da, semg_a)
        hrb = pltpu.async_copy(rs_hbm.at[sb], rb, semg_b)
        hdb = pltpu.async_copy(rs_hbm.at[dbc], db, semg_b)
        hra.wait()
        hda.wait()
        _out_compute(ra, da, ova, e4, c4, KB // 4)
        hoa = pltpu.async_copy(ova, eo_hbm.at[pl.ds(base * D_OUT, KB * D_OUT)],
                               semo_a)
        hrb.wait()
        hdb.wait()
        _out_compute(rb, db, ovb, e4, c4, KB // 4)
        hob = pltpu.async_copy(
            ovb, eo_hbm.at[pl.ds((base + KB) * D_OUT, KB * D_OUT)], semo_b)
        hoa.wait()
        hob.wait()
        return carry

    lax.fori_loop(0, NPAIR, pair, 0)

    tb = base0 + TBASE
    st = ivs.at[pl.ds(0, KT)]
    dtc = ivd.at[pl.ds(0, KT)]
    pltpu.sync_copy(ef_hbm.at[pl.ds(tb, KT)], st)
    pltpu.sync_copy(ef_hbm.at[pl.ds(E + tb, KT)], dtc)
    hr = pltpu.async_copy(rs_hbm.at[st], rt, semg_a)
    hd = pltpu.async_copy(rs_hbm.at[dtc], dt, semg_a)
    hr.wait()
    hd.wait()
    _out_compute(rt, dt, ova, e4, c4, KT // 4)
    pltpu.sync_copy(ova.at[pl.ds(0, KT * D_OUT)],
                    eo_hbm.at[pl.ds(tb * D_OUT, KT * D_OUT)])


def _make_sc_kernels():
    mesh = plsc.VectorSubcoreMesh(core_axis_name="c", subcore_axis_name="s")
    cp = pltpu.CompilerParams(needs_layout_passes=False,
                              use_tc_tiling_on_sc=False)
    edge_msg = pl.kernel(
        _edge_msg_body,
        out_type=jax.ShapeDtypeStruct((NC * NPAD, D_H), jnp.float32),
        mesh=mesh,
        compiler_params=cp,
        scratch_types=[
            pltpu.VMEM((2 * KB,), jnp.int32),
            pltpu.VMEM((2 * KB,), jnp.int32),
            pltpu.VMEM((KB, D_H), jnp.float32),
            pltpu.VMEM((KB, D_H), jnp.float32),
            pltpu.VMEM((KB, D_H), jnp.float32),
            pltpu.VMEM((KB, D_H), jnp.float32),
            pltpu.VMEM((KT, D_H), jnp.float32),
            pltpu.VMEM((KT, D_H), jnp.float32),
            pltpu.VMEM_SHARED((NPAD, D_H), jnp.float32),
            pltpu.SemaphoreType.DMA,
            pltpu.SemaphoreType.DMA,
            pltpu.SemaphoreType.DMA,
            pltpu.SemaphoreType.DMA,
        ],
    )
    edge_out = pl.kernel(
        _edge_out_body,
        out_type=[
            jax.ShapeDtypeStruct((E * D_OUT,), jnp.float32),
            jax.ShapeDtypeStruct((NPAD, 2 * D_OUT), jnp.float32),
        ],
        mesh=mesh,
        compiler_params=cp,
        scratch_types=[
            pltpu.VMEM((8, 8), jnp.float32),
            pltpu.VMEM((2 * KB,), jnp.int32),
            pltpu.VMEM((2 * KB,), jnp.int32),
            pltpu.VMEM((KB, 2 * D_OUT), jnp.float32),
            pltpu.VMEM((KB, 2 * D_OUT), jnp.float32),
            pltpu.VMEM((KB, 2 * D_OUT), jnp.float32),
            pltpu.VMEM((KB, 2 * D_OUT), jnp.float32),
            pltpu.VMEM((KB * D_OUT,), jnp.float32),
            pltpu.VMEM((KB * D_OUT,), jnp.float32),
            pltpu.VMEM((KT, 2 * D_OUT), jnp.float32),
            pltpu.VMEM((KT, 2 * D_OUT), jnp.float32),
            pltpu.SemaphoreType.DMA,
            pltpu.SemaphoreType.DMA,
            pltpu.SemaphoreType.DMA,
            pltpu.SemaphoreType.DMA,
        ],
    )
    return edge_msg, edge_out


def kernel(x, edge_index, W_in, b_in, W1, b1, W2, b2, We, be):
    edge_msg, edge_out = _make_sc_kernels()
    ef = edge_index.reshape(2 * E)
    p, q, rsx = _node_pre(x, W_in, b_in.reshape(1, -1), W1, b1.reshape(1, -1),
                          W2, b2.reshape(1, -1), We, be.reshape(1, -1))
    zeros = jnp.zeros((NPAD, D_H), jnp.float32)
    hp = edge_msg(p, q, ef, zeros)
    wcat = jnp.concatenate([We[0:8], We[136:144]], axis=1)
    eo, _ = edge_out(hp, rsx, wcat, ef)
    return eo.reshape(E, D_OUT)


# speedup vs baseline: 8.2261x; 1.0213x over previous
import jax
import jax.numpy as jnp
from jax import lax
from jax.experimental import pallas as pl
from jax.experimental.pallas import tpu as pltpu
from jax.experimental.pallas import tpu_sc as plsc

N = 10000
E = 320000
D_IN = 128
D_H = 8
D_OUT = 4
NC = 2
NS = 16
NW = NC * NS
EPW = E // NW
K = 80
NCHUNK = EPW // K
NPAD = 10240
RPT = NPAD // NS
KB = 128
NFULL = EPW // KB
NPAIR = NFULL // 2
KT = EPW - NFULL * KB
TBASE = NFULL * KB
KB2 = 80
NQUAD = EPW // (4 * KB2)


def _node_pre_body(x_ref, win_ref, bin_ref, w1_ref, b1_ref, w2_ref, b2_ref,
                   we_ref, be_ref, p_ref, q_ref, rsx_ref):
    x = x_ref[...]
    w2 = w2_ref[...]
    wc = jnp.dot(w1_ref[...], w2, preferred_element_type=jnp.float32)
    bc = jnp.dot(b1_ref[...], w2, preferred_element_type=jnp.float32) + b2_ref[...]
    h = jnp.dot(x, win_ref[...], preferred_element_type=jnp.float32) + bin_ref[...]
    h = jnp.where(h > 0, h, jnp.exp(h) - 1.0)
    hx = jnp.concatenate([h, x], axis=1)
    p = jnp.dot(hx, wc[:136] - wc[136:], preferred_element_type=jnp.float32) + bc
    q = jnp.dot(hx, wc[136:], preferred_element_type=jnp.float32)
    we = we_ref[...]
    rx = jnp.dot(x, we[8:136], preferred_element_type=jnp.float32)
    sx = jnp.dot(x, we[144:], preferred_element_type=jnp.float32) + be_ref[...]
    pad = jnp.zeros((NPAD - N, D_H), jnp.float32)
    p_ref[...] = jnp.concatenate([p, pad], axis=0)
    q_ref[...] = jnp.concatenate([q, pad], axis=0)
    rsx_ref[...] = jnp.concatenate([jnp.concatenate([rx, sx], axis=1), pad],
                                   axis=0)


_node_pre = pl.pallas_call(
    _node_pre_body,
    out_shape=[
        jax.ShapeDtypeStruct((NPAD, D_H), jnp.float32),
        jax.ShapeDtypeStruct((NPAD, D_H), jnp.float32),
        jax.ShapeDtypeStruct((NPAD, 2 * D_OUT), jnp.float32),
    ],
)


def _msg_compute(pv, qv, rowp, colp, ngroups):
    for j in range(ngroups):
        ridx = rowp + 2 * j
        pvec = plsc.load_gather(pv, [ridx, colp])
        qvec = plsc.load_gather(qv, [ridx, colp])
        v = pvec + qvec
        t = jnp.where(v > 0, v, jnp.exp(v) - 1.0)
        plsc.store_scatter(pv, [ridx, colp], t)


def _edge_msg_body(p_hbm, q_hbm, ef_hbm, zero_hbm, out_hbm,
                   ivdx, ivsx, ivdy, ivsy,
                   pxa, qxa, pxb, qxb, pya, qya, pyb, qyb, acc,
                   semi, semgx, semgy, semsx, semsy):
    c = lax.axis_index("c")
    s = lax.axis_index("s")
    wid = c * NS + s
    sl = pl.ds(s * RPT, RPT)
    pltpu.sync_copy(zero_hbm.at[sl], acc.at[sl])
    plsc.subcore_barrier()

    iota = lax.iota(jnp.int32, 16)
    rowp = lax.shift_right_arithmetic(iota, jnp.int32(3))
    colp = lax.bitwise_and(iota, jnp.int32(7))
    base0 = wid * EPW
    ng = KB2 * D_H // 16

    def quad(t, carry):
        bx = base0 + t * (4 * KB2)
        by = bx + 2 * KB2
        hix = [pltpu.async_copy(ef_hbm.at[pl.ds(E + bx, 2 * KB2)], ivdx, semi),
               pltpu.async_copy(ef_hbm.at[pl.ds(bx, 2 * KB2)], ivsx, semi)]
        hiy = [pltpu.async_copy(ef_hbm.at[pl.ds(E + by, 2 * KB2)], ivdy, semi),
               pltpu.async_copy(ef_hbm.at[pl.ds(by, 2 * KB2)], ivsy, semi)]
        for h in hix:
            h.wait()
        dxa = ivdx.at[pl.ds(0, KB2)]
        dxb = ivdx.at[pl.ds(KB2, KB2)]
        gx = [pltpu.async_copy(p_hbm.at[dxa], pxa, semgx),
              pltpu.async_copy(q_hbm.at[ivsx.at[pl.ds(0, KB2)]], qxa, semgx),
              pltpu.async_copy(p_hbm.at[dxb], pxb, semgx),
              pltpu.async_copy(q_hbm.at[ivsx.at[pl.ds(KB2, KB2)]], qxb, semgx)]
        for h in hiy:
            h.wait()
        dya = ivdy.at[pl.ds(0, KB2)]
        dyb = ivdy.at[pl.ds(KB2, KB2)]
        gy = [pltpu.async_copy(p_hbm.at[dya], pya, semgy),
              pltpu.async_copy(q_hbm.at[ivsy.at[pl.ds(0, KB2)]], qya, semgy),
              pltpu.async_copy(p_hbm.at[dyb], pyb, semgy),
              pltpu.async_copy(q_hbm.at[ivsy.at[pl.ds(KB2, KB2)]], qyb, semgy)]
        for h in gx:
            h.wait()
        _msg_compute(pxa, qxa, rowp, colp, ng)
        sxa = pltpu.async_copy(pxa, acc.at[dxa], semsx, add=True)
        _msg_compute(pxb, qxb, rowp, colp, ng)
        sxb = pltpu.async_copy(pxb, acc.at[dxb], semsx, add=True)
        for h in gy:
            h.wait()
        _msg_compute(pya, qya, rowp, colp, ng)
        sya = pltpu.async_copy(pya, acc.at[dya], semsy, add=True)
        _msg_compute(pyb, qyb, rowp, colp, ng)
        syb = pltpu.async_copy(pyb, acc.at[dyb], semsy, add=True)
        sxa.wait()
        sxb.wait()
        sya.wait()
        syb.wait()
        return carry

    lax.fori_loop(0, NQUAD, quad, 0)

    fb = base0 + NQUAD * 4 * KB2
    dt = ivdx.at[pl.ds(0, KB2)]
    st = ivsx.at[pl.ds(0, KB2)]
    pltpu.sync_copy(ef_hbm.at[pl.ds(E + fb, KB2)], dt)
    pltpu.sync_copy(ef_hbm.at[pl.ds(fb, KB2)], st)
    hp = pltpu.async_copy(p_hbm.at[dt], pxa, semgx)
    hq = pltpu.async_copy(q_hbm.at[st], qxa, semgx)
    hp.wait()
    hq.wait()
    _msg_compute(pxa, qxa, rowp, colp, ng)
    pltpu.sync_copy(pxa, acc.at[dt], add=True)

    plsc.subcore_barrier()
    pltpu.sync_copy(acc.at[sl], out_hbm.at[pl.ds(c * NPAD + s * RPT, RPT)])


def _out_compute(rb, db, ov, e4, c4, ngroups):
    for g in range(ngroups):
        ridx = e4 + 4 * g
        rvec = plsc.load_gather(rb, [ridx, c4])
        svec = plsc.load_gather(db, [ridx, c4 + 4])
        ov[pl.ds(g * 16, 16)] = rvec + svec


def _edge_out_body(hp_hbm, rsx_hbm, wcat_hbm, ef_hbm, eo_hbm, rs_hbm,
                   wv, ivs, ivd, ra, da, rb, db,
                   ova, ovb, rt, dt, semg_a, semg_b, semo_a, semo_b):
    c = lax.axis_index("c")
    s = lax.axis_index("s")
    wid = c * NS + s
    iota = lax.iota(jnp.int32, 16)
    rowp = lax.shift_right_arithmetic(iota, jnp.int32(3))
    colp = lax.bitwise_and(iota, jnp.int32(7))
    e4 = lax.shift_right_arithmetic(iota, jnp.int32(2))
    c4 = lax.bitwise_and(iota, jnp.int32(3))
    base0 = wid * EPW

    pltpu.sync_copy(wcat_hbm, wv)
    wks = [plsc.load_gather(wv, [jnp.full((16,), k, jnp.int32), colp])
           for k in range(8)]
    r0 = s * RPT

    def rs_block(i, carry):
        rr = r0 + i * KB
        h0 = pltpu.async_copy(hp_hbm.at[pl.ds(rr, KB)], ra, semg_a)
        h1 = pltpu.async_copy(hp_hbm.at[pl.ds(NPAD + rr, KB)], da, semg_b)
        hx = pltpu.async_copy(rsx_hbm.at[pl.ds(rr, KB)], rb, semo_a)
        h0.wait()
        h1.wait()
        for j in range(KB * D_H // 16):
            ridx = rowp + 2 * j
            v = plsc.load_gather(ra, [ridx, colp]) + \
                plsc.load_gather(da, [ridx, colp])
            plsc.store_scatter(ra, [ridx, colp], v)
        hx.wait()
        for g in range(KB * D_H // 16):
            nid = rowp + 2 * g
            acc = plsc.load_gather(rb, [nid, colp])
            for k in range(8):
                hk = plsc.load_gather(ra, [nid, jnp.full((16,), k, jnp.int32)])
                acc = acc + hk * wks[k]
            plsc.store_scatter(rb, [nid, colp], acc)
        pltpu.sync_copy(rb, rs_hbm.at[pl.ds(rr, KB)])
        return carry

    lax.fori_loop(0, RPT // KB, rs_block, 0)
    plsc.subcore_barrier()


    def pair(t, carry):
        base = base0 + t * (2 * KB)
        pltpu.sync_copy(ef_hbm.at[pl.ds(base, 2 * KB)], ivs)
        pltpu.sync_copy(ef_hbm.at[pl.ds(E + base, 2 * KB)], ivd)
        sa = ivs.at[pl.ds(0, KB)]
        sb = ivs.at[pl.ds(KB, KB)]
        dac = ivd.at[pl.ds(0, KB)]
        dbc = ivd.at[pl.ds(KB, KB)]
        hra = pltpu.async_copy(rs_hbm.at[sa], ra, semg_a)
        hda = pltpu.async_copy(rs_hbm.at[dac], da, semg_a)
        hrb = pltpu.async_copy(rs_hbm.at[sb], rb, semg_b)
        hdb = pltpu.async_copy(rs_hbm.at[dbc], db, semg_b)
        hra.wait()
        hda.wait()
        _out_compute(ra, da, ova, e4, c4, KB // 4)
        hoa = pltpu.async_copy(ova, eo_hbm.at[pl.ds(base * D_OUT, KB * D_OUT)],
                               semo_a)
        hrb.wait()
        hdb.wait()
        _out_compute(rb, db, ovb, e4, c4, KB // 4)
        hob = pltpu.async_copy(
            ovb, eo_hbm.at[pl.ds((base + KB) * D_OUT, KB * D_OUT)], semo_b)
        hoa.wait()
        hob.wait()
        return carry

    lax.fori_loop(0, NPAIR, pair, 0)

    tb = base0 + TBASE
    st = ivs.at[pl.ds(0, KT)]
    dtc = ivd.at[pl.ds(0, KT)]
    pltpu.sync_copy(ef_hbm.at[pl.ds(tb, KT)], st)
    pltpu.sync_copy(ef_hbm.at[pl.ds(E + tb, KT)], dtc)
    hr = pltpu.async_copy(rs_hbm.at[st], rt, semg_a)
    hd = pltpu.async_copy(rs_hbm.at[dtc], dt, semg_a)
    hr.wait()
    hd.wait()
    _out_compute(rt, dt, ova, e4, c4, KT // 4)
    pltpu.sync_copy(ova.at[pl.ds(0, KT * D_OUT)],
                    eo_hbm.at[pl.ds(tb * D_OUT, KT * D_OUT)])


def _make_sc_kernels():
    mesh = plsc.VectorSubcoreMesh(core_axis_name="c", subcore_axis_name="s")
    cp = pltpu.CompilerParams(needs_layout_passes=False,
                              use_tc_tiling_on_sc=False)
    edge_msg = pl.kernel(
        _edge_msg_body,
        out_type=jax.ShapeDtypeStruct((NC * NPAD, D_H), jnp.float32),
        mesh=mesh,
        compiler_params=cp,
        scratch_types=[
            pltpu.VMEM((2 * KB2,), jnp.int32),
            pltpu.VMEM((2 * KB2,), jnp.int32),
            pltpu.VMEM((2 * KB2,), jnp.int32),
            pltpu.VMEM((2 * KB2,), jnp.int32),
            pltpu.VMEM((KB2, D_H), jnp.float32),
            pltpu.VMEM((KB2, D_H), jnp.float32),
            pltpu.VMEM((KB2, D_H), jnp.float32),
            pltpu.VMEM((KB2, D_H), jnp.float32),
            pltpu.VMEM((KB2, D_H), jnp.float32),
            pltpu.VMEM((KB2, D_H), jnp.float32),
            pltpu.VMEM((KB2, D_H), jnp.float32),
            pltpu.VMEM((KB2, D_H), jnp.float32),
            pltpu.VMEM_SHARED((NPAD, D_H), jnp.float32),
            pltpu.SemaphoreType.DMA,
            pltpu.SemaphoreType.DMA,
            pltpu.SemaphoreType.DMA,
            pltpu.SemaphoreType.DMA,
            pltpu.SemaphoreType.DMA,
        ],
    )
    edge_out = pl.kernel(
        _edge_out_body,
        out_type=[
            jax.ShapeDtypeStruct((E * D_OUT,), jnp.float32),
            jax.ShapeDtypeStruct((NPAD, 2 * D_OUT), jnp.float32),
        ],
        mesh=mesh,
        compiler_params=cp,
        scratch_types=[
            pltpu.VMEM((8, 8), jnp.float32),
            pltpu.VMEM((2 * KB,), jnp.int32),
            pltpu.VMEM((2 * KB,), jnp.int32),
            pltpu.VMEM((KB, 2 * D_OUT), jnp.float32),
            pltpu.VMEM((KB, 2 * D_OUT), jnp.float32),
            pltpu.VMEM((KB, 2 * D_OUT), jnp.float32),
            pltpu.VMEM((KB, 2 * D_OUT), jnp.float32),
            pltpu.VMEM((KB * D_OUT,), jnp.float32),
            pltpu.VMEM((KB * D_OUT,), jnp.float32),
            pltpu.VMEM((KT, 2 * D_OUT), jnp.float32),
            pltpu.VMEM((KT, 2 * D_OUT), jnp.float32),
            pltpu.SemaphoreType.DMA,
            pltpu.SemaphoreType.DMA,
            pltpu.SemaphoreType.DMA,
            pltpu.SemaphoreType.DMA,
        ],
    )
    return edge_msg, edge_out


def kernel(x, edge_index, W_in, b_in, W1, b1, W2, b2, We, be):
    edge_msg, edge_out = _make_sc_kernels()
    ef = edge_index.reshape(2 * E)
    p, q, rsx = _node_pre(x, W_in, b_in.reshape(1, -1), W1, b1.reshape(1, -1),
                          W2, b2.reshape(1, -1), We, be.reshape(1, -1))
    zeros = jnp.zeros((NPAD, D_H), jnp.float32)
    hp = edge_msg(p, q, ef, zeros)
    wcat = jnp.concatenate([We[0:8], We[136:144]], axis=1)
    eo, _ = edge_out(hp, rsx, wcat, ef)
    return eo.reshape(E, D_OUT)
